# Initial kernel scaffold; baseline (speedup 1.0000x reference)
import jax
import jax.numpy as jnp
from jax.experimental import pallas as pl


def kernel(qk_dots, relative_attention_bias):
    raise NotImplementedError("write your pallas kernel here")


# speedup vs baseline: 24.9210x; 24.9210x over previous
import math

import jax
import jax.numpy as jnp
import numpy as np
from jax.experimental import pallas as pl
from jax.experimental.pallas import tpu as pltpu

_BLK = 256


def _bias_segments(seq_q, seq_k, num_buckets=32, max_distance=128):
    d = np.arange(-(seq_q - 1), seq_k)
    n = -d
    offs = (n < 0).astype(np.int32) * (num_buckets // 2)
    n = np.abs(n)
    max_exact = num_buckets // 4
    val = max_exact + (
        np.log(np.maximum(n.astype(np.float32), np.float32(1e-20)) / np.float32(max_exact))
        / np.float32(math.log(max_distance / max_exact))
        * np.float32(num_buckets // 2 - max_exact)
    ).astype(np.int32)
    val = np.minimum(val, num_buckets // 2 - 1)
    bucket = offs + np.where(n < max_exact, n, val)
    segs = []
    start = int(d[0])
    cur = int(bucket[0])
    for k in range(1, len(d)):
        if int(bucket[k]) != cur:
            segs.append((start, int(d[k - 1]), cur))
            start = int(d[k])
            cur = int(bucket[k])
    segs.append((start, int(d[-1]), cur))
    return segs


def kernel(qk_dots, relative_attention_bias):
    batch, heads, seq_q, seq_k = qk_dots.shape
    assert batch == 1 and seq_q % _BLK == 0 and seq_k % _BLK == 0
    ti = seq_q // _BLK
    tj = seq_k // _BLK
    nd = ti + tj - 1

    segs = _bias_segments(seq_q, seq_k, relative_attention_bias.shape[0])

    qk = qk_dots.reshape(heads, seq_q, seq_k)
    tbl = relative_attention_bias.T

    def body(tbl_ref, qk_ref, out_ref, slab_ref):
        h = pl.program_id(0)
        i = pl.program_id(1)
        j = pl.program_id(2)

        @pl.when(jnp.logical_and(i == 0, j == 0))
        def _build_slab():
            ai = jax.lax.broadcasted_iota(jnp.int32, (_BLK, _BLK), 0)
            bi = jax.lax.broadcasted_iota(jnp.int32, (_BLK, _BLK), 1)
            dmat = bi - ai
            for t in range(nd):
                off = (t - (ti - 1)) * _BLK
                lo = off - (_BLK - 1)
                hi = off + (_BLK - 1)
                tsegs = [s for s in segs if s[1] >= lo and s[0] <= hi]
                acc = jnp.full((_BLK, _BLK), tbl_ref[h, tsegs[0][2]], jnp.float32)
                for (ds_, _de, b_) in tsegs[1:]:
                    acc = jnp.where(dmat >= (ds_ - off), tbl_ref[h, b_], acc)
                slab_ref[t] = acc

        out_ref[0] = qk_ref[0] + slab_ref[j - i + (ti - 1)]

    out = pl.pallas_call(
        body,
        grid=(heads, ti, tj),
        in_specs=[
            pl.BlockSpec(memory_space=pltpu.SMEM),
            pl.BlockSpec((1, _BLK, _BLK), lambda h, i, j: (h, i, j)),
        ],
        out_specs=pl.BlockSpec((1, _BLK, _BLK), lambda h, i, j: (h, i, j)),
        out_shape=jax.ShapeDtypeStruct((heads, seq_q, seq_k), jnp.float32),
        scratch_shapes=[pltpu.VMEM((nd, _BLK, _BLK), jnp.float32)],
    )(tbl, qk)
    return out.reshape(batch, heads, seq_q, seq_k)


# speedup vs baseline: 82.3425x; 3.3041x over previous
import math

import jax
import jax.numpy as jnp
import numpy as np
from jax.experimental import pallas as pl
from jax.experimental.pallas import tpu as pltpu

_BLK = 256


def _bias_segments(seq_q, seq_k, num_buckets=32, max_distance=128):
    d = np.arange(-(seq_q - 1), seq_k)
    n = -d
    offs = (n < 0).astype(np.int32) * (num_buckets // 2)
    n = np.abs(n)
    max_exact = num_buckets // 4
    val = max_exact + (
        np.log(np.maximum(n.astype(np.float32), np.float32(1e-20)) / np.float32(max_exact))
        / np.float32(math.log(max_distance / max_exact))
        * np.float32(num_buckets // 2 - max_exact)
    ).astype(np.int32)
    val = np.minimum(val, num_buckets // 2 - 1)
    bucket = offs + np.where(n < max_exact, n, val)
    segs = []
    start = int(d[0])
    cur = int(bucket[0])
    for k in range(1, len(d)):
        if int(bucket[k]) != cur:
            segs.append((start, int(d[k - 1]), cur))
            start = int(d[k])
            cur = int(bucket[k])
    segs.append((start, int(d[-1]), cur))
    return segs


def kernel(qk_dots, relative_attention_bias):
    batch, heads, seq_q, seq_k = qk_dots.shape
    assert batch == 1 and seq_q % _BLK == 0 and seq_k % _BLK == 0
    ti = seq_q // _BLK
    tj = seq_k // _BLK
    nd = ti + tj - 1

    segs = _bias_segments(seq_q, seq_k, relative_attention_bias.shape[0])

    qk = qk_dots.reshape(heads, seq_q, seq_k)
    tbl = relative_attention_bias.T

    def body(tbl_ref, qk_ref, out_ref, slab_ref):
        h = pl.program_id(0)
        i = pl.program_id(1)

        @pl.when(i == 0)
        def _build_slab():
            ai = jax.lax.broadcasted_iota(jnp.int32, (_BLK, _BLK), 0)
            bi = jax.lax.broadcasted_iota(jnp.int32, (_BLK, _BLK), 1)
            dmat = bi - ai
            for t in range(nd):
                off = (t - (ti - 1)) * _BLK
                lo = off - (_BLK - 1)
                hi = off + (_BLK - 1)
                tsegs = [s for s in segs if s[1] >= lo and s[0] <= hi]
                acc = jnp.full((_BLK, _BLK), tbl_ref[h, tsegs[0][2]], jnp.float32)
                for (ds_, _de, b_) in tsegs[1:]:
                    acc = jnp.where(dmat >= (ds_ - off), tbl_ref[h, b_], acc)
                slab_ref[t] = acc

        for jt in range(tj):
            sl = slice(jt * _BLK, (jt + 1) * _BLK)
            out_ref[0, :, sl] = qk_ref[0, :, sl] + slab_ref[jt - i + (ti - 1)]

    out = pl.pallas_call(
        body,
        grid=(heads, ti),
        in_specs=[
            pl.BlockSpec(memory_space=pltpu.SMEM),
            pl.BlockSpec((1, _BLK, seq_k), lambda h, i: (h, i, 0)),
        ],
        out_specs=pl.BlockSpec((1, _BLK, seq_k), lambda h, i: (h, i, 0)),
        out_shape=jax.ShapeDtypeStruct((heads, seq_q, seq_k), jnp.float32),
        scratch_shapes=[pltpu.VMEM((nd, _BLK, _BLK), jnp.float32)],
    )(tbl, qk)
    return out.reshape(batch, heads, seq_q, seq_k)


# speedup vs baseline: 93.8232x; 1.1394x over previous
import math

import jax
import jax.numpy as jnp
import numpy as np
from jax.experimental import pallas as pl
from jax.experimental.pallas import tpu as pltpu

_BLK = 512


def _bias_segments(seq_q, seq_k, num_buckets=32, max_distance=128):
    d = np.arange(-(seq_q - 1), seq_k)
    n = -d
    offs = (n < 0).astype(np.int32) * (num_buckets // 2)
    n = np.abs(n)
    max_exact = num_buckets // 4
    val = max_exact + (
        np.log(np.maximum(n.astype(np.float32), np.float32(1e-20)) / np.float32(max_exact))
        / np.float32(math.log(max_distance / max_exact))
        * np.float32(num_buckets // 2 - max_exact)
    ).astype(np.int32)
    val = np.minimum(val, num_buckets // 2 - 1)
    bucket = offs + np.where(n < max_exact, n, val)
    segs = []
    start = int(d[0])
    cur = int(bucket[0])
    for k in range(1, len(d)):
        if int(bucket[k]) != cur:
            segs.append((start, int(d[k - 1]), cur))
            start = int(d[k])
            cur = int(bucket[k])
    segs.append((start, int(d[-1]), cur))
    return segs


def kernel(qk_dots, relative_attention_bias):
    batch, heads, seq_q, seq_k = qk_dots.shape
    assert batch == 1 and seq_q % _BLK == 0 and seq_k % _BLK == 0
    ti = seq_q // _BLK
    tj = seq_k // _BLK
    nd = ti + tj - 1

    segs = _bias_segments(seq_q, seq_k, relative_attention_bias.shape[0])

    qk = qk_dots.reshape(heads, seq_q, seq_k)
    tbl = relative_attention_bias.T

    def body(tbl_ref, qk_ref, out_ref, slab_ref):
        h = pl.program_id(0)
        i = pl.program_id(1)

        @pl.when(i == 0)
        def _build_slab():
            ai = jax.lax.broadcasted_iota(jnp.int32, (_BLK, _BLK), 0)
            bi = jax.lax.broadcasted_iota(jnp.int32, (_BLK, _BLK), 1)
            dmat = bi - ai
            for t in range(nd):
                off = (t - (ti - 1)) * _BLK
                lo = off - (_BLK - 1)
                hi = off + (_BLK - 1)
                tsegs = [s for s in segs if s[1] >= lo and s[0] <= hi]
                acc = jnp.full((_BLK, _BLK), tbl_ref[h, tsegs[0][2]], jnp.float32)
                for (ds_, _de, b_) in tsegs[1:]:
                    acc = jnp.where(dmat >= (ds_ - off), tbl_ref[h, b_], acc)
                slab_ref[t] = acc

        for jt in range(tj):
            sl = slice(jt * _BLK, (jt + 1) * _BLK)
            out_ref[0, :, sl] = qk_ref[0, :, sl] + slab_ref[jt - i + (ti - 1)]

    out = pl.pallas_call(
        body,
        grid=(heads, ti),
        in_specs=[
            pl.BlockSpec(memory_space=pltpu.SMEM),
            pl.BlockSpec((1, _BLK, seq_k), lambda h, i: (h, i, 0)),
        ],
        out_specs=pl.BlockSpec((1, _BLK, seq_k), lambda h, i: (h, i, 0)),
        out_shape=jax.ShapeDtypeStruct((heads, seq_q, seq_k), jnp.float32),
        scratch_shapes=[pltpu.VMEM((nd, _BLK, _BLK), jnp.float32)],
    )(tbl, qk)
    return out.reshape(batch, heads, seq_q, seq_k)


# speedup vs baseline: 96.9564x; 1.0334x over previous
import math

import jax
import jax.numpy as jnp
import numpy as np
from jax.experimental import pallas as pl
from jax.experimental.pallas import tpu as pltpu

_BLK = 1024


def _bias_segments(seq_q, seq_k, num_buckets=32, max_distance=128):
    d = np.arange(-(seq_q - 1), seq_k)
    n = -d
    offs = (n < 0).astype(np.int32) * (num_buckets // 2)
    n = np.abs(n)
    max_exact = num_buckets // 4
    val = max_exact + (
        np.log(np.maximum(n.astype(np.float32), np.float32(1e-20)) / np.float32(max_exact))
        / np.float32(math.log(max_distance / max_exact))
        * np.float32(num_buckets // 2 - max_exact)
    ).astype(np.int32)
    val = np.minimum(val, num_buckets // 2 - 1)
    bucket = offs + np.where(n < max_exact, n, val)
    segs = []
    start = int(d[0])
    cur = int(bucket[0])
    for k in range(1, len(d)):
        if int(bucket[k]) != cur:
            segs.append((start, int(d[k - 1]), cur))
            start = int(d[k])
            cur = int(bucket[k])
    segs.append((start, int(d[-1]), cur))
    return segs


def kernel(qk_dots, relative_attention_bias):
    batch, heads, seq_q, seq_k = qk_dots.shape
    assert batch == 1 and seq_q % _BLK == 0 and seq_k % _BLK == 0
    ti = seq_q // _BLK
    tj = seq_k // _BLK
    nd = ti + tj - 1

    segs = _bias_segments(seq_q, seq_k, relative_attention_bias.shape[0])

    qk = qk_dots.reshape(heads, seq_q, seq_k)
    tbl = relative_attention_bias.T

    def body(tbl_ref, qk_ref, out_ref, slab_ref):
        h = pl.program_id(0)
        i = pl.program_id(1)

        @pl.when(i == 0)
        def _build_slab():
            ai = jax.lax.broadcasted_iota(jnp.int32, (_BLK, _BLK), 0)
            bi = jax.lax.broadcasted_iota(jnp.int32, (_BLK, _BLK), 1)
            dmat = bi - ai
            for t in range(nd):
                off = (t - (ti - 1)) * _BLK
                lo = off - (_BLK - 1)
                hi = off + (_BLK - 1)
                tsegs = [s for s in segs if s[1] >= lo and s[0] <= hi]
                acc = jnp.full((_BLK, _BLK), tbl_ref[h, tsegs[0][2]], jnp.float32)
                for (ds_, _de, b_) in tsegs[1:]:
                    acc = jnp.where(dmat >= (ds_ - off), tbl_ref[h, b_], acc)
                slab_ref[t] = acc

        for jt in range(tj):
            sl = slice(jt * _BLK, (jt + 1) * _BLK)
            out_ref[0, :, sl] = qk_ref[0, :, sl] + slab_ref[jt - i + (ti - 1)]

    out = pl.pallas_call(
        body,
        grid=(heads, ti),
        in_specs=[
            pl.BlockSpec(memory_space=pltpu.SMEM),
            pl.BlockSpec((1, _BLK, seq_k), lambda h, i: (h, i, 0)),
        ],
        out_specs=pl.BlockSpec((1, _BLK, seq_k), lambda h, i: (h, i, 0)),
        out_shape=jax.ShapeDtypeStruct((heads, seq_q, seq_k), jnp.float32),
        scratch_shapes=[pltpu.VMEM((nd, _BLK, _BLK), jnp.float32)],
    )(tbl, qk)
    return out.reshape(batch, heads, seq_q, seq_k)


# speedup vs baseline: 97.2964x; 1.0035x over previous
import math

import jax
import jax.numpy as jnp
import numpy as np
from jax.experimental import pallas as pl
from jax.experimental.pallas import tpu as pltpu

_BLK = 1024
_SUB = 256


def _bias_segments(seq_q, seq_k, num_buckets=32, max_distance=128):
    d = np.arange(-(seq_q - 1), seq_k)
    n = -d
    offs = (n < 0).astype(np.int32) * (num_buckets // 2)
    n = np.abs(n)
    max_exact = num_buckets // 4
    val = max_exact + (
        np.log(np.maximum(n.astype(np.float32), np.float32(1e-20)) / np.float32(max_exact))
        / np.float32(math.log(max_distance / max_exact))
        * np.float32(num_buckets // 2 - max_exact)
    ).astype(np.int32)
    val = np.minimum(val, num_buckets // 2 - 1)
    bucket = offs + np.where(n < max_exact, n, val)
    segs = []
    start = int(d[0])
    cur = int(bucket[0])
    for k in range(1, len(d)):
        if int(bucket[k]) != cur:
            segs.append((start, int(d[k - 1]), cur))
            start = int(d[k])
            cur = int(bucket[k])
    segs.append((start, int(d[-1]), cur))
    return segs


def kernel(qk_dots, relative_attention_bias):
    batch, heads, seq_q, seq_k = qk_dots.shape
    assert batch == 1 and seq_q % _BLK == 0 and seq_k % _SUB == 0
    ti = seq_q // _BLK
    si = seq_q // _SUB
    sj = seq_k // _SUB
    rpb = _BLK // _SUB
    nd = si + sj - 1

    segs = _bias_segments(seq_q, seq_k, relative_attention_bias.shape[0])

    qk = qk_dots.reshape(heads, seq_q, seq_k)
    tbl = relative_attention_bias.T

    def body(tbl_ref, qk_ref, out_ref, slab_ref):
        h = pl.program_id(0)
        i = pl.program_id(1)

        @pl.when(i == 0)
        def _build_slab():
            ai = jax.lax.broadcasted_iota(jnp.int32, (_SUB, _SUB), 0)
            bi = jax.lax.broadcasted_iota(jnp.int32, (_SUB, _SUB), 1)
            dmat = bi - ai
            for t in range(nd):
                off = (t - (si - 1)) * _SUB
                lo = off - (_SUB - 1)
                hi = off + (_SUB - 1)
                tsegs = [s for s in segs if s[1] >= lo and s[0] <= hi]
                acc = jnp.full((_SUB, _SUB), tbl_ref[h, tsegs[0][2]], jnp.float32)
                for (ds_, _de, b_) in tsegs[1:]:
                    acc = jnp.where(dmat >= (ds_ - off), tbl_ref[h, b_], acc)
                slab_ref[t] = acc

        for it in range(rpb):
            rsl = slice(it * _SUB, (it + 1) * _SUB)
            for jt in range(sj):
                csl = slice(jt * _SUB, (jt + 1) * _SUB)
                t_dyn = jt - (i * rpb + it) + (si - 1)
                out_ref[0, rsl, csl] = qk_ref[0, rsl, csl] + slab_ref[t_dyn]

    out = pl.pallas_call(
        body,
        grid=(heads, ti),
        in_specs=[
            pl.BlockSpec(memory_space=pltpu.SMEM),
            pl.BlockSpec((1, _BLK, seq_k), lambda h, i: (h, i, 0)),
        ],
        out_specs=pl.BlockSpec((1, _BLK, seq_k), lambda h, i: (h, i, 0)),
        out_shape=jax.ShapeDtypeStruct((heads, seq_q, seq_k), jnp.float32),
        scratch_shapes=[pltpu.VMEM((nd, _SUB, _SUB), jnp.float32)],
    )(tbl, qk)
    return out.reshape(batch, heads, seq_q, seq_k)
